# baseline (speedup 1.0000x reference)
import functools

import jax
import jax.numpy as jnp
from jax import lax
from jax.experimental import pallas as pl
from jax.experimental.pallas import tpu as pltpu
from jax.experimental.pallas import tpu_sc as plsc

B = 8
L = 512

NC = 2
NS = 16
NW = NC * NS
ROWS_PER_TILE = (B * L) // NW
SUB = 64
NSUB = ROWS_PER_TILE // SUB


def _softmax_kernel(p_ref, q_ref):
    P = p_ref[...]
    mx = jnp.max(P, axis=1, keepdims=True)
    e = jnp.exp(P - mx)
    q_ref[...] = 1.0 - e / jnp.sum(e, axis=1, keepdims=True)


def _sc_gather_kernel(q_hbm, xs_hbm, ys_hbm, m_hbm, idx_v, ys_v, rows_v,
                      out_v, sem):
    wid = lax.axis_index("s") * NC + lax.axis_index("c")
    b = wid // (NW // B)
    pltpu.sync_copy(ys_hbm.at[pl.ds(b * L, L)], ys_v)
    for sub in range(NSUB):
        base = wid * ROWS_PER_TILE + sub * SUB
        pltpu.sync_copy(xs_hbm.at[pl.ds(base, SUB)], idx_v)
        pltpu.async_copy(q_hbm.at[idx_v], rows_v, sem).wait()

        def body(i, _):
            i16 = jnp.full((16,), i, jnp.int32)
            for g in range(L // 16):
                lanes = lax.iota(jnp.int32, 16) + (g * 16)
                ysg = ys_v[pl.ds(g * 16, 16)]
                vals = plsc.load_gather(rows_v, [i16, ysg])
                plsc.store_scatter(out_v, [i16, lanes], vals)
            return 0

        lax.fori_loop(0, SUB, body, 0)
        pltpu.sync_copy(out_v, m_hbm.at[pl.ds(base, SUB)])


@functools.partial(
    pl.kernel,
    out_type=jax.ShapeDtypeStruct((B * L, L), jnp.float32),
    mesh=plsc.VectorSubcoreMesh(core_axis_name="c", subcore_axis_name="s"),
    compiler_params=pltpu.CompilerParams(needs_layout_passes=False),
    scratch_types=[
        pltpu.VMEM((SUB,), jnp.int32),
        pltpu.VMEM((L,), jnp.int32),
        pltpu.VMEM((SUB, L), jnp.float32),
        pltpu.VMEM((SUB, L), jnp.float32),
        pltpu.SemaphoreType.DMA,
    ],
)
def _sc_gather(q_hbm, xs_hbm, ys_hbm, m_hbm, idx_v, ys_v, rows_v, out_v, sem):
    _sc_gather_kernel(q_hbm, xs_hbm, ys_hbm, m_hbm, idx_v, ys_v, rows_v,
                      out_v, sem)


def _dp_kernel(xs_ref, ys_ref, xlen_ref, ylen_ref, q_ref, m_ref, dn_ref,
               out_ref, t_ref):
    f32 = jnp.float32
    Q = q_ref[...]
    dn0 = dn_ref[0]
    dn1 = dn_ref[1]

    iota_s = lax.broadcasted_iota(jnp.int32, (L, L), 0)
    iota_l = lax.broadcasted_iota(jnp.int32, (L, L), 1)
    qdn = jnp.sum(jnp.where(iota_s == dn0, Q, 0.0), axis=0, keepdims=True)
    qcol = jnp.sum(jnp.where(iota_l == dn1, Q, 0.0), axis=1, keepdims=True)

    cy_rows = []
    cx_rows = []
    for b in range(B):
        xb = xs_ref[pl.ds(b, 1), :]
        ohxT = (jnp.broadcast_to(xb, (L, L)) == iota_s).astype(f32)
        yb = ys_ref[pl.ds(b, 1), :]
        ohyT = (jnp.broadcast_to(yb, (L, L)) == iota_s).astype(f32)
        cyb = lax.dot_general(qdn, ohyT, (((1,), (0,)), ((), ())),
                              preferred_element_type=f32)
        cy_rows.append(cyb)
        cx_rows.append(lax.dot_general(qcol, ohxT, (((0,), (0,)), ((), ())),
                                       preferred_element_type=f32))
        mb = m_ref[pl.ds(b * L, L), :]
        skewed = pltpu.roll(mb - cyb, 0, 1, stride=1, stride_axis=0)
        tb = jnp.transpose(skewed)
        t_ref[:, pl.ds(b, 1), :] = tb.reshape(L, 1, L)

    cy_all = jnp.concatenate(cy_rows, axis=0)
    cxl = jnp.concatenate(cx_rows, axis=0)

    lane = lax.broadcasted_iota(jnp.int32, (B, L), 1)
    xe = xlen_ref[...] - 1
    ye = ylen_ref[...] - 1
    de = xe + ye
    lanekx = lane == jnp.broadcast_to(xe - 1, (B, L))
    ccy_mask = lane <= jnp.broadcast_to(ye - 1, (B, L))
    ccy_at = jnp.sum(jnp.where(ccy_mask, cy_all, 0.0), axis=1, keepdims=True)

    inf = jnp.float32(jnp.inf)

    def one_diag(d, f_prev, r_prev, acc):
        c = (d - 2) & (L - 1)
        tslab = t_ref[pl.ds(c, 1), :, :].reshape(B, L)
        r1 = pltpu.roll(f_prev, 1, 1)
        b0 = jnp.where(d <= L + 1, jnp.float32(0.0), inf)
        r1 = jnp.where(lane == 0, b0, r1)
        f_new = jnp.minimum(jnp.minimum(r_prev + tslab, f_prev), r1 + cxl)
        hit = lanekx & jnp.broadcast_to(de == d, (B, L))
        acc = jnp.where(hit, f_new, acc)
        return f_new, r1, acc

    U = 4

    def step(s, carry):
        f_prev, r_prev, acc = carry
        for r in range(1, U + 1):
            f_prev, r_prev, acc = one_diag(U * s + r, f_prev, r_prev, acc)
        return f_prev, r_prev, acc

    n_steps = (jnp.max(de) + (U - 1)) // U
    init = (jnp.full((B, L), inf, f32), jnp.full((B, L), inf, f32),
            jnp.zeros((B, L), f32))
    _, _, acc = lax.fori_loop(0, n_steps, step, init)

    phi = jnp.sum(acc, axis=1, keepdims=True)
    total = jnp.sum(phi + ccy_at, axis=0, keepdims=True)
    out_ref[...] = total * (1.0 / B)


@jax.jit
def kernel(xs, ys, x_lengths, y_lengths, P, do_nothing_ij):
    Q = pl.pallas_call(
        _softmax_kernel,
        out_shape=jax.ShapeDtypeStruct((L, L), jnp.float32),
        in_specs=[pl.BlockSpec((L, L), lambda: (0, 0))],
        out_specs=pl.BlockSpec((L, L), lambda: (0, 0)),
    )(P)
    M = _sc_gather(Q, xs.reshape(-1), ys.reshape(-1))
    out = pl.pallas_call(
        _dp_kernel,
        out_shape=jax.ShapeDtypeStruct((1, 1), jnp.float32),
        in_specs=[
            pl.BlockSpec((B, L), lambda: (0, 0)),
            pl.BlockSpec((B, L), lambda: (0, 0)),
            pl.BlockSpec((B, 1), lambda: (0, 0)),
            pl.BlockSpec((B, 1), lambda: (0, 0)),
            pl.BlockSpec((L, L), lambda: (0, 0)),
            pl.BlockSpec((B * L, L), lambda: (0, 0)),
            pl.BlockSpec(memory_space=pltpu.SMEM),
        ],
        out_specs=pl.BlockSpec((1, 1), lambda: (0, 0)),
        scratch_shapes=[
            pltpu.VMEM((L, B, L), jnp.float32),
        ],
    )(xs, ys, x_lengths.reshape(B, 1), y_lengths.reshape(B, 1), Q, M,
      do_nothing_ij)
    return out[0, 0]


# speedup vs baseline: 1.3637x; 1.3637x over previous
import functools

import jax
import jax.numpy as jnp
from jax import lax
from jax.experimental import pallas as pl
from jax.experimental.pallas import tpu as pltpu
from jax.experimental.pallas import tpu_sc as plsc

B = 8
L = 512

NC = 2
NS = 16
NW = NC * NS
ROWS_PER_TILE = (B * L) // NW
SUB = 64
NSUB = ROWS_PER_TILE // SUB


def _softmax_kernel(p_ref, q_ref):
    P = p_ref[...]
    mx = jnp.max(P, axis=1, keepdims=True)
    e = jnp.exp(P - mx)
    q_ref[...] = 1.0 - e / jnp.sum(e, axis=1, keepdims=True)


def _sc_gather_kernel(q_hbm, xs_hbm, ys_hbm, m_hbm, idx_v, ys_v, rows_v,
                      out_v, sem):
    wid = lax.axis_index("s") * NC + lax.axis_index("c")
    b = wid // (NW // B)
    pltpu.sync_copy(ys_hbm.at[pl.ds(b * L, L)], ys_v)
    for sub in range(NSUB):
        base = wid * ROWS_PER_TILE + sub * SUB
        pltpu.sync_copy(xs_hbm.at[pl.ds(base, SUB)], idx_v)
        pltpu.async_copy(q_hbm.at[idx_v], rows_v, sem).wait()

        @plsc.parallel_loop(0, SUB)
        def body(i):
            i16 = jnp.full((16,), i, jnp.int32)
            for g in range(L // 16):
                lanes = lax.iota(jnp.int32, 16) + (g * 16)
                ysg = ys_v[pl.ds(g * 16, 16)]
                vals = plsc.load_gather(rows_v, [i16, ysg])
                plsc.store_scatter(out_v, [i16, lanes], vals)
        pltpu.sync_copy(out_v, m_hbm.at[pl.ds(base, SUB)])


@functools.partial(
    pl.kernel,
    out_type=jax.ShapeDtypeStruct((B * L, L), jnp.float32),
    mesh=plsc.VectorSubcoreMesh(core_axis_name="c", subcore_axis_name="s"),
    compiler_params=pltpu.CompilerParams(needs_layout_passes=False),
    scratch_types=[
        pltpu.VMEM((SUB,), jnp.int32),
        pltpu.VMEM((L,), jnp.int32),
        pltpu.VMEM((SUB, L), jnp.float32),
        pltpu.VMEM((SUB, L), jnp.float32),
        pltpu.SemaphoreType.DMA,
    ],
)
def _sc_gather(q_hbm, xs_hbm, ys_hbm, m_hbm, idx_v, ys_v, rows_v, out_v, sem):
    _sc_gather_kernel(q_hbm, xs_hbm, ys_hbm, m_hbm, idx_v, ys_v, rows_v,
                      out_v, sem)


def _dp_kernel(xs_ref, ys_ref, xlen_ref, ylen_ref, q_ref, m_ref, dn_ref,
               out_ref, t_ref):
    f32 = jnp.float32
    Q = q_ref[...]
    dn0 = dn_ref[0]
    dn1 = dn_ref[1]

    iota_s = lax.broadcasted_iota(jnp.int32, (L, L), 0)
    iota_l = lax.broadcasted_iota(jnp.int32, (L, L), 1)
    qdn = jnp.sum(jnp.where(iota_s == dn0, Q, 0.0), axis=0, keepdims=True)
    qcol = jnp.sum(jnp.where(iota_l == dn1, Q, 0.0), axis=1, keepdims=True)

    cy_rows = []
    cx_rows = []
    for b in range(B):
        xb = xs_ref[pl.ds(b, 1), :]
        ohxT = (jnp.broadcast_to(xb, (L, L)) == iota_s).astype(f32)
        yb = ys_ref[pl.ds(b, 1), :]
        ohyT = (jnp.broadcast_to(yb, (L, L)) == iota_s).astype(f32)
        cyb = lax.dot_general(qdn, ohyT, (((1,), (0,)), ((), ())),
                              preferred_element_type=f32)
        cy_rows.append(cyb)
        cx_rows.append(lax.dot_general(qcol, ohxT, (((0,), (0,)), ((), ())),
                                       preferred_element_type=f32))
        mb = m_ref[pl.ds(b * L, L), :]
        skewed = pltpu.roll(mb - cyb, 0, 1, stride=1, stride_axis=0)
        tb = jnp.transpose(skewed)
        t_ref[:, pl.ds(b, 1), :] = tb.reshape(L, 1, L)

    cy_all = jnp.concatenate(cy_rows, axis=0)
    cxl = jnp.concatenate(cx_rows, axis=0)

    lane = lax.broadcasted_iota(jnp.int32, (B, L), 1)
    xe = xlen_ref[...] - 1
    ye = ylen_ref[...] - 1
    de = xe + ye
    lanekx = lane == jnp.broadcast_to(xe - 1, (B, L))
    ccy_mask = lane <= jnp.broadcast_to(ye - 1, (B, L))
    ccy_at = jnp.sum(jnp.where(ccy_mask, cy_all, 0.0), axis=1, keepdims=True)

    inf = jnp.float32(jnp.inf)

    def one_diag(d, f_prev, r_prev, acc):
        c = (d - 2) & (L - 1)
        tslab = t_ref[pl.ds(c, 1), :, :].reshape(B, L)
        r1 = pltpu.roll(f_prev, 1, 1)
        b0 = jnp.where(d <= L + 1, jnp.float32(0.0), inf)
        r1 = jnp.where(lane == 0, b0, r1)
        f_new = jnp.minimum(jnp.minimum(r_prev + tslab, f_prev), r1 + cxl)
        hit = lanekx & jnp.broadcast_to(de == d, (B, L))
        acc = jnp.where(hit, f_new, acc)
        return f_new, r1, acc

    U = 4

    def step(s, carry):
        f_prev, r_prev, acc = carry
        for r in range(1, U + 1):
            f_prev, r_prev, acc = one_diag(U * s + r, f_prev, r_prev, acc)
        return f_prev, r_prev, acc

    n_steps = (jnp.max(de) + (U - 1)) // U
    init = (jnp.full((B, L), inf, f32), jnp.full((B, L), inf, f32),
            jnp.zeros((B, L), f32))
    _, _, acc = lax.fori_loop(0, n_steps, step, init)

    phi = jnp.sum(acc, axis=1, keepdims=True)
    total = jnp.sum(phi + ccy_at, axis=0, keepdims=True)
    out_ref[...] = total * (1.0 / B)


@jax.jit
def kernel(xs, ys, x_lengths, y_lengths, P, do_nothing_ij):
    Q = pl.pallas_call(
        _softmax_kernel,
        out_shape=jax.ShapeDtypeStruct((L, L), jnp.float32),
        in_specs=[pl.BlockSpec((L, L), lambda: (0, 0))],
        out_specs=pl.BlockSpec((L, L), lambda: (0, 0)),
    )(P)
    M = _sc_gather(Q, xs.reshape(-1), ys.reshape(-1))
    out = pl.pallas_call(
        _dp_kernel,
        out_shape=jax.ShapeDtypeStruct((1, 1), jnp.float32),
        in_specs=[
            pl.BlockSpec((B, L), lambda: (0, 0)),
            pl.BlockSpec((B, L), lambda: (0, 0)),
            pl.BlockSpec((B, 1), lambda: (0, 0)),
            pl.BlockSpec((B, 1), lambda: (0, 0)),
            pl.BlockSpec((L, L), lambda: (0, 0)),
            pl.BlockSpec((B * L, L), lambda: (0, 0)),
            pl.BlockSpec(memory_space=pltpu.SMEM),
        ],
        out_specs=pl.BlockSpec((1, 1), lambda: (0, 0)),
        scratch_shapes=[
            pltpu.VMEM((L, B, L), jnp.float32),
        ],
    )(xs, ys, x_lengths.reshape(B, 1), y_lengths.reshape(B, 1), Q, M,
      do_nothing_ij)
    return out[0, 0]
